# baseline (speedup 1.0000x reference)
import functools

import jax
import jax.numpy as jnp
from jax import lax
from jax.experimental import pallas as pl
from jax.experimental.pallas import tpu as pltpu
from jax.experimental.pallas import tpu_sc as plsc

NC, NS, LANES = 2, 16, 16
N_LIB = 500
N_PAD = 512
D_EMB = 512
BATCH = 8
TOPK = 5
IMG = 112
BIG = 3.0e38

_MESH = plsc.VectorSubcoreMesh(
    core_axis_name="c", subcore_axis_name="s", num_cores=NC, num_subcores=NS
)


def _lane_iota():
    return lax.iota(jnp.int32, 16)


def _splat_i32(x):
    return jnp.full((16,), 0, jnp.int32) + x


def _topk_body(zt_hbm, zf_hbm, idx_out, val_out,
               zt_v, z_v, dloc_v, dv, io_v, vo_v, dsh):
    cid = lax.axis_index("c")
    sid = lax.axis_index("s")

    @pl.when(cid == 0)
    def _core0():
        e0 = sid * 32
        pltpu.sync_copy(zt_hbm.at[:, pl.ds(e0, 32)], zt_v)
        pltpu.sync_copy(zf_hbm, z_v)

        def body(c, accs):
            zl0 = zt_v[c, pl.ds(0, 16)]
            zl1 = zt_v[c, pl.ds(16, 16)]
            new = []
            for b in range(BATCH):
                zb = plsc.load_gather(z_v, [_splat_i32(b * D_EMB + c)])
                d0 = zb - zl0
                d1 = zb - zl1
                new.append(accs[2 * b] + d0 * d0)
                new.append(accs[2 * b + 1] + d1 * d1)
            return tuple(new)

        accs = lax.fori_loop(
            0, D_EMB, body,
            tuple(jnp.zeros((16,), jnp.float32) for _ in range(2 * BATCH)))

        lane = _lane_iota()
        for b in range(BATCH):
            for g in range(2):
                e_g = e0 + g * 16 + lane
                v = jnp.where(e_g < N_LIB, accs[2 * b + g],
                              jnp.float32(BIG))
                dloc_v[b, pl.ds(g * 16, 16)] = v
        pltpu.sync_copy(dloc_v, dsh.at[:, pl.ds(e0, 32)])
        plsc.subcore_barrier()

        @pl.when(sid < BATCH)
        def _topk():
            b = sid
            pltpu.sync_copy(dsh.at[b], dv)
            io_vec = jnp.zeros((16,), jnp.int32)
            vo_vec = jnp.zeros((16,), jnp.float32)
            for k in range(TOPK):
                def scan(j, carry):
                    bv, bi = carry
                    x = dv[pl.ds(j * 16, 16)]
                    ii = j * 16 + lane
                    m = x < bv
                    return (jnp.where(m, x, bv), jnp.where(m, ii, bi))

                bv, bi = lax.fori_loop(
                    0, N_PAD // 16, scan,
                    (jnp.full((16,), BIG, jnp.float32),
                     jnp.zeros((16,), jnp.int32)))
                minv = jnp.min(bv)
                cand = jnp.where(bv == minv, bi, jnp.int32(1 << 30))
                mini = jnp.min(cand)
                io_vec = jnp.where(lane == k, _splat_i32(mini), io_vec)
                vo_vec = jnp.where(lane == k,
                                   jnp.zeros((16,), jnp.float32) + minv,
                                   vo_vec)
                plsc.store_scatter(dv, [_splat_i32(mini)],
                                   jnp.full((16,), BIG, jnp.float32),
                                   mask=lane == 0)
            io_v[...] = io_vec
            vo_v[...] = vo_vec
            pltpu.sync_copy(io_v, idx_out.at[pl.ds(b * 16, 16)])
            pltpu.sync_copy(vo_v, val_out.at[pl.ds(b * 16, 16)])


_topk_call = functools.partial(
    pl.kernel,
    out_type=(jax.ShapeDtypeStruct((BATCH * 16,), jnp.int32),
              jax.ShapeDtypeStruct((BATCH * 16,), jnp.float32)),
    mesh=_MESH,
    compiler_params=pltpu.CompilerParams(use_tc_tiling_on_sc=False, needs_layout_passes=False),
    scratch_types=(
        pltpu.VMEM((D_EMB, 32), jnp.float32),
        pltpu.VMEM((BATCH * D_EMB,), jnp.float32),
        pltpu.VMEM((BATCH, 32), jnp.float32),
        pltpu.VMEM((N_PAD,), jnp.float32),
        pltpu.VMEM((16,), jnp.int32),
        pltpu.VMEM((16,), jnp.float32),
        pltpu.VMEM_SHARED((BATCH, N_PAD), jnp.float32),
    ),
)(_topk_body)


def _smap_task(idx_v, g_v, acc_v, sem, m_hbm, f_hbm, out_hbm, t,
               m5, fv, n_rows, segs, s_len, out_len):
    lane = _lane_iota()
    b = t // TOPK
    k_ = t % TOPK
    base = plsc.load_gather(idx_v, [_splat_i32(b * 16 + k_)]) * n_rows
    for j in range(n_rows // 16):
        g_v[pl.ds(j * 16, 16)] = base + (j * 16 + lane)
    desc = pltpu.async_copy(m_hbm.at[g_v], m5, sem)
    pltpu.sync_copy(f_hbm.at[b], fv)
    desc.wait()

    offs = [(o, False) for o in range(0, (s_len // 16) * 16, 16)]
    if s_len % 16:
        offs.append((s_len - 16, True))

    for off, tail in offs:
        mtails = [jnp.full((16,), seg * s_len + off, jnp.int32) + lane
                  for seg in range(segs)]

        def body(r, acc):
            for seg in range(segs):
                c = r * segs + seg
                if segs == 1 and not tail:
                    mv = m5[r, pl.ds(off, 16)]
                else:
                    mv = plsc.load_gather(m5, [_splat_i32(r), mtails[seg]])
                if tail:
                    fvv = plsc.load_gather(
                        fv, [_splat_i32(c), jnp.full((16,), off, jnp.int32) + lane])
                else:
                    fvv = fv[c, pl.ds(off, 16)]
                d = mv - fvv
                acc = acc + d * d
            return acc

        acc = lax.fori_loop(0, n_rows, body, jnp.zeros((16,), jnp.float32))
        if tail:
            plsc.store_scatter(acc_v, [jnp.full((16,), off, jnp.int32) + lane],
                               acc)
        else:
            acc_v[pl.ds(off, 16)] = acc
    pltpu.sync_copy(acc_v.at[pl.ds(0, out_len)],
                    out_hbm.at[pl.ds(t * out_len, out_len)])


def _smap_body(idxf_hbm, m1_hbm, m2_hbm, m3_hbm, f1_hbm, f2_hbm, f3_hbm,
               s1_out, s2_out, s3_out,
               idx_v, g64_v, g32_v, g16_v, acc_v, sem):
    cid = lax.axis_index("c")
    sid = lax.axis_index("s")
    w = sid * NC + cid
    pltpu.sync_copy(idxf_hbm, idx_v)

    def s1_task(t):
        def go(m5, fv):
            _smap_task(idx_v, g64_v, acc_v, sem, m1_hbm, f1_hbm, s1_out, t,
                       m5, fv, n_rows=64, segs=1, s_len=784, out_len=784)
        pl.run_scoped(go, pltpu.VMEM((64, 784), jnp.float32),
                      pltpu.VMEM((64, 784), jnp.float32))

    def s2_task(t):
        def go(m5, fv):
            _smap_task(idx_v, g32_v, acc_v, sem, m2_hbm, f2_hbm, s2_out, t,
                       m5, fv, n_rows=32, segs=4, s_len=196, out_len=200)
        pl.run_scoped(go, pltpu.VMEM((32, 784), jnp.float32),
                      pltpu.VMEM((128, 196), jnp.float32))

    def s3_task(t):
        def go(m5, fv):
            _smap_task(idx_v, g16_v, acc_v, sem, m3_hbm, f3_hbm, s3_out, t,
                       m5, fv, n_rows=16, segs=16, s_len=49, out_len=56)
        pl.run_scoped(go, pltpu.VMEM((16, 784), jnp.float32),
                      pltpu.VMEM((256, 49), jnp.float32))

    s1_task(w)
    pl.when(w >= 24)(lambda: s1_task(w + 8))
    pl.when(w < 24)(lambda: s2_task(w))
    pl.when(w < 16)(lambda: s2_task(w + 24))
    pl.when(w >= 16)(lambda: s3_task(w - 16))
    pl.when(w < 24)(lambda: s3_task(w + 16))


_smap_call = functools.partial(
    pl.kernel,
    out_type=(jax.ShapeDtypeStruct((BATCH * TOPK * 784,), jnp.float32),
              jax.ShapeDtypeStruct((BATCH * TOPK * 200,), jnp.float32),
              jax.ShapeDtypeStruct((BATCH * TOPK * 56,), jnp.float32)),
    mesh=_MESH,
    compiler_params=pltpu.CompilerParams(use_tc_tiling_on_sc=False, needs_layout_passes=False),
    scratch_types=(
        pltpu.VMEM((128,), jnp.int32),
        pltpu.VMEM((64,), jnp.int32),
        pltpu.VMEM((32,), jnp.int32),
        pltpu.VMEM((16,), jnp.int32),
        pltpu.VMEM((784,), jnp.float32),
        pltpu.SemaphoreType.DMA,
    ),
)(_smap_body)


def _resize_body(vals_ref, s1_ref, s2_ref, s3_ref, r1_ref, r2_ref, r3_ref,
                 zs_ref, out_ref):
    zs_ref[...] = jnp.mean(jnp.sqrt(vals_ref[...]), axis=1, keepdims=True)
    total = jnp.zeros((BATCH, IMG, IMG), jnp.float32)
    for s_ref, r_ref in ((s1_ref, r1_ref), (s2_ref, r2_ref),
                         (s3_ref, r3_ref)):
        s = jnp.min(s_ref[...], axis=1)
        r = r_ref[...]
        t = lax.dot_general(s, r, (((1,), (1,)), ((), ())),
                            preferred_element_type=jnp.float32,
                            precision=lax.Precision.HIGHEST)
        o = lax.dot_general(t, r, (((1,), (1,)), ((), ())),
                            preferred_element_type=jnp.float32,
                            precision=lax.Precision.HIGHEST)
        total = total + o
    out_ref[...] = total.reshape(BATCH, 1, IMG, IMG)


def _resize_mat(src):
    return jax.image.resize(jnp.eye(src, dtype=jnp.float32), (IMG, src),
                            method="bilinear")


def kernel(z, z_lib, f1, f2, f3, m1, m2, m3):
    zt = jnp.pad(z_lib.T, ((0, 0), (0, N_PAD - N_LIB)))
    zf = z.reshape(BATCH * D_EMB)

    idx8, valsq8 = _topk_call(zt, zf)

    m1v = m1.reshape(N_LIB * 64, 784)
    m2v = m2.reshape(N_LIB * 32, 784)
    m3v = m3.reshape(N_LIB * 16, 784)
    f1v = f1.reshape(BATCH, 64, 784)
    f2v = f2.reshape(BATCH, 128, 196)
    f3v = f3.reshape(BATCH, 256, 49)

    s1p, s2p, s3p = _smap_call(idx8.reshape(128), m1v, m2v, m3v,
                               f1v, f2v, f3v)

    s1 = s1p.reshape(BATCH, TOPK, 28, 28)
    s2 = s2p.reshape(BATCH, TOPK, 200)[:, :, :196].reshape(BATCH, TOPK, 14, 14)
    s3 = s3p.reshape(BATCH, TOPK, 56)[:, :, :49].reshape(BATCH, TOPK, 7, 7)
    vals = valsq8.reshape(BATCH, 16)[:, :TOPK]

    z_score, smap = pl.pallas_call(
        _resize_body,
        out_shape=(jax.ShapeDtypeStruct((BATCH, 1), jnp.float32),
                   jax.ShapeDtypeStruct((BATCH, 1, IMG, IMG), jnp.float32)),
    )(vals, s1, s2, s3, _resize_mat(28), _resize_mat(14), _resize_mat(7))
    return z_score, smap
